# Initial kernel scaffold; baseline (speedup 1.0000x reference)
import jax
import jax.numpy as jnp
from jax.experimental import pallas as pl


def kernel(xi, W):
    raise NotImplementedError("write your pallas kernel here")


# speedup vs baseline: 1.3587x; 1.3587x over previous
import functools

import jax
import jax.numpy as jnp
from jax import lax
from jax.experimental import pallas as pl
from jax.experimental.pallas import tpu as pltpu
from jax.experimental.pallas import tpu_sc as plsc

Q = 7
VOCAB = 1025
D = 1024
T = 16384
NC, NS = 2, 16
NW = NC * NS
TPW = T // NW
C = 16
NCH = TPW // C
NGRP = NCH // 2
LANES = 16


def _make_embed_call():
    mesh = plsc.VectorSubcoreMesh(core_axis_name="c", subcore_axis_name="s")

    @functools.partial(
        pl.kernel,
        out_type=jax.ShapeDtypeStruct((T, D), jnp.float32),
        mesh=mesh,
        scratch_types=[
            pltpu.VMEM((Q, TPW), jnp.int32),
            pltpu.VMEM((C, D), jnp.float32),
            pltpu.VMEM((C, D), jnp.float32),
            pltpu.VMEM((C, D), jnp.float32),
            pltpu.VMEM((C, D), jnp.float32),
            pltpu.SemaphoreType.DMA,
            pltpu.SemaphoreType.DMA,
            pltpu.SemaphoreType.DMA,
            pltpu.SemaphoreType.DMA,
        ],
    )
    def embed(w_hbm, idx_hbm, out_hbm,
              idx_v, acc0, acc1, ring0, ring1, gsem0, gsem1, osem0, osem1):
        accs = (acc0, acc1)
        rings = (ring0, ring1)
        gsems = (gsem0, gsem1)
        osems = (osem0, osem1)

        wid = lax.axis_index("s") * NC + lax.axis_index("c")
        base = wid * TPW

        pltpu.sync_copy(idx_hbm.at[wid], idx_v)

        def gather_desc(k, cc, rs):
            idx_sl = idx_v.at[k, pl.ds(cc * C, C)]
            return pltpu.make_async_copy(w_hbm.at[idx_sl], rings[rs], gsems[rs])

        def out_desc(cc, p):
            return pltpu.make_async_copy(
                accs[p], out_hbm.at[pl.ds(base + cc * C, C)], osems[p])

        def accum(src, dst, first):
            def row_body(r, _):
                for j in range(D // LANES):
                    sl = pl.ds(j * LANES, LANES)
                    v = src[r, sl]
                    if first:
                        dst[r, sl] = v
                    else:
                        plsc.addupdate(dst.at[r, sl], v)
                return 0
            lax.fori_loop(0, C, row_body, 0)

        gather_desc(0, 0, 0).start()

        def group_body(g, _):
            c0 = g * 2
            for b in range(2):
                cc = c0 + b
                for k in range(Q):
                    s = b * Q + k
                    rs = s % 2
                    ns = s + 1
                    nk = ns % Q
                    ncc = c0 + ns // Q
                    if ns // Q < 2:
                        gather_desc(nk, ncc, ns % 2).start()
                    else:
                        @pl.when(g + 1 < NGRP)
                        def _():
                            gather_desc(0, c0 + 2, 0).start()
                    gather_desc(k, cc, rs).wait()
                    if k == 0:
                        @pl.when(g >= 1)
                        def _():
                            out_desc(cc, b).wait()
                    accum(rings[rs], accs[b], first=(k == 0))
                out_desc(cc, b).start()
            return 0

        lax.fori_loop(0, NGRP, group_body, 0)

        out_desc(NCH - 2, 0).wait()
        out_desc(NCH - 1, 1).wait()

    return embed


_embed = _make_embed_call()


def kernel(xi, W):
    idx = xi[:, :Q].astype(jnp.int32) + (jnp.arange(Q, dtype=jnp.int32) * VOCAB)[None, :]
    idx_all = idx.T.reshape(Q, NW, TPW).transpose(1, 0, 2)
    w_flat = W.reshape(W.shape[0] * VOCAB, D)
    return _embed(w_flat, idx_all)


# speedup vs baseline: 2.3850x; 1.7554x over previous
import functools

import jax
import jax.numpy as jnp
from jax import lax
from jax.experimental import pallas as pl
from jax.experimental.pallas import tpu as pltpu
from jax.experimental.pallas import tpu_sc as plsc

Q = 7
VOCAB = 1025
D = 1024
T = 16384
NC, NS = 2, 16
NW = NC * NS
TPW = T // NW
C = 16
NCH = TPW // C
NGRP = NCH // 2
LANES = 16


def _make_embed_call():
    mesh = plsc.VectorSubcoreMesh(core_axis_name="c", subcore_axis_name="s")

    @functools.partial(
        pl.kernel,
        out_type=jax.ShapeDtypeStruct((T, D), jnp.float32),
        mesh=mesh,
        scratch_types=[
            pltpu.VMEM((Q, TPW), jnp.int32),
            pltpu.VMEM((C, D), jnp.float32),
            pltpu.VMEM((C, D), jnp.float32),
            pltpu.VMEM((C, D), jnp.float32),
            pltpu.VMEM((C, D), jnp.float32),
            pltpu.SemaphoreType.DMA,
            pltpu.SemaphoreType.DMA,
            pltpu.SemaphoreType.DMA,
            pltpu.SemaphoreType.DMA,
        ],
    )
    def embed(w_hbm, idx_hbm, out_hbm,
              idx_v, acc0, acc1, ring0, ring1, gsem0, gsem1, osem0, osem1):
        accs = (acc0, acc1)
        rings = (ring0, ring1)
        gsems = (gsem0, gsem1)
        osems = (osem0, osem1)

        wid = lax.axis_index("s") * NC + lax.axis_index("c")
        base = wid * TPW

        pltpu.sync_copy(idx_hbm.at[wid], idx_v)

        def gather_desc(k, cc, rs):
            idx_sl = idx_v.at[k, pl.ds(cc * C, C)]
            return pltpu.make_async_copy(w_hbm.at[idx_sl], rings[rs], gsems[rs])

        def out_desc(cc, p):
            return pltpu.make_async_copy(
                accs[p], out_hbm.at[pl.ds(base + cc * C, C)], osems[p])

        def accum(src, dst, first):
            @plsc.parallel_loop(0, C * (D // LANES), 1, unroll=8)
            def _(i):
                r = lax.shift_right_logical(i, 6)
                col = pl.multiple_of(
                    lax.shift_left(jnp.bitwise_and(i, D // LANES - 1), 4), LANES)
                sl = pl.ds(col, LANES)
                v = src[r, sl]
                if first:
                    dst[r, sl] = v
                else:
                    plsc.addupdate(dst.at[r, sl], v)

        gather_desc(0, 0, 0).start()

        def group_body(g, _):
            c0 = g * 2
            for b in range(2):
                cc = c0 + b
                for k in range(Q):
                    s = b * Q + k
                    rs = s % 2
                    ns = s + 1
                    nk = ns % Q
                    ncc = c0 + ns // Q
                    if ns // Q < 2:
                        gather_desc(nk, ncc, ns % 2).start()
                    else:
                        @pl.when(g + 1 < NGRP)
                        def _():
                            gather_desc(0, c0 + 2, 0).start()
                    gather_desc(k, cc, rs).wait()
                    if k == 0:
                        @pl.when(g >= 1)
                        def _():
                            out_desc(cc, b).wait()
                    accum(rings[rs], accs[b], first=(k == 0))
                out_desc(cc, b).start()
            return 0

        lax.fori_loop(0, NGRP, group_body, 0)

        out_desc(NCH - 2, 0).wait()
        out_desc(NCH - 1, 1).wait()

    return embed


_embed = _make_embed_call()


def kernel(xi, W):
    idx = xi[:, :Q].astype(jnp.int32) + (jnp.arange(Q, dtype=jnp.int32) * VOCAB)[None, :]
    idx_all = idx.T.reshape(Q, NW, TPW).transpose(1, 0, 2)
    w_flat = W.reshape(W.shape[0] * VOCAB, D)
    return _embed(w_flat, idx_all)
